# baseline (speedup 1.0000x reference)
import functools

import jax
import jax.numpy as jnp
from jax.experimental import pallas as pl
from jax.experimental.pallas import tpu as pltpu

_T = 512
_K = 4
_SB = 256


def _router_kernel(x_ref, wr_ref, br_ref, slot_ref, gate_ref, te_ref,
                   nv_ref, *, n_tiles, tile):
    N = x_ref.shape[0]
    E = wr_ref.shape[0]
    G, T = n_tiles, tile
    x = x_ref[...]
    wr = wr_ref[...]
    logits = jax.lax.dot_general(
        x, wr, (((1,), (1,)), ((), ())), preferred_element_type=jnp.float32)
    logits = logits + br_ref[...]
    m = jnp.max(logits, axis=1, keepdims=True)
    s = jnp.sum(jnp.exp(logits - m), axis=1, keepdims=True)
    gate_ref[...] = 1.0 / s
    idx = jnp.argmax(logits, axis=1).astype(jnp.int32)

    lane_e = jax.lax.broadcasted_iota(jnp.int32, (N, E), 1)
    oh = (lane_e == idx[:, None]).astype(jnp.float32)

    r = jax.lax.broadcasted_iota(jnp.int32, (N, N), 0)
    c = jax.lax.broadcasted_iota(jnp.int32, (N, N), 1)
    tri = (r > c).astype(jnp.float32)
    pos_e = jax.lax.dot_general(
        tri, oh, (((1,), (0,)), ((), ())),
        preferred_element_type=jnp.float32)
    pos = jnp.sum(oh * pos_e, axis=1, keepdims=True)

    counts = jnp.sum(oh, axis=0, keepdims=True)
    tiles_e = jnp.floor((counts + (T - 1)) * (1.0 / T))
    ue = jax.lax.broadcasted_iota(jnp.int32, (E, E), 0)
    uc = jax.lax.broadcasted_iota(jnp.int32, (E, E), 1)
    ut = (ue <= uc).astype(jnp.float32)
    cum_tiles = jax.lax.dot_general(
        tiles_e, ut, (((1,), (0,)), ((), ())),
        preferred_element_type=jnp.float32)
    tile_row_off = (cum_tiles - tiles_e) * T
    slot = jnp.sum(oh * tile_row_off, axis=1, keepdims=True) + pos
    slot_ref[...] = slot.astype(jnp.int32)

    total_tiles = jnp.sum(tiles_e)
    lane_e1 = jax.lax.broadcasted_iota(jnp.int32, (1, E), 1).astype(jnp.float32)
    last_e = jnp.max(lane_e1 * (counts > 0))
    gi = jax.lax.broadcasted_iota(jnp.int32, (G, 1), 0).astype(jnp.float32)
    te_raw = jnp.sum((cum_tiles <= gi).astype(jnp.float32),
                     axis=1, keepdims=True)
    te = jnp.minimum(te_raw, last_e)
    te_ref[...] = te.astype(jnp.int32)

    oh_te = (te == jax.lax.broadcasted_iota(jnp.int32, (G, E), 1)
             .astype(jnp.float32)).astype(jnp.float32)
    cum_te = jnp.sum(oh_te * cum_tiles, axis=1, keepdims=True)
    tiles_te = jnp.sum(oh_te * tiles_e, axis=1, keepdims=True)
    counts_te = jnp.sum(oh_te * counts, axis=1, keepdims=True)
    local_t = gi - (cum_te - tiles_te)
    nv = jnp.clip(counts_te - local_t * T, 0.0, float(T))
    nv = jnp.where(gi < total_tiles, nv, 0.0)
    nv_ref[...] = nv.astype(jnp.int32)


def _ffn_kernel(te_ref, nv_ref, xp_ref, w1_ref, b1_ref, w2_ref, b2_ref,
                out_ref, *, n_chunks, tile, sub):
    del te_ref
    i = pl.program_id(0)
    k = pl.program_id(1)
    nv = nv_ref[i]

    @pl.when(nv > 0)
    def _():
        w1 = w1_ref[0].astype(jnp.bfloat16)
        w2 = w2_ref[0].astype(jnp.bfloat16)
        for sb in range(tile // sub):
            rows = pl.ds(sb * sub, sub)

            @pl.when(sb * sub < nv)
            def _():
                xb = xp_ref[rows, :].astype(jnp.bfloat16)
                h = jax.lax.dot_general(
                    xb, w1, (((1,), (1,)), ((), ())),
                    preferred_element_type=jnp.float32)
                h = jnp.maximum(h + b1_ref[0], 0.0).astype(jnp.bfloat16)
                part = jax.lax.dot_general(
                    h, w2, (((1,), (1,)), ((), ())),
                    preferred_element_type=jnp.float32)

                @pl.when(k == 0)
                def _():
                    out_ref[rows, :] = part

                @pl.when((k > 0) & (k < n_chunks - 1))
                def _():
                    out_ref[rows, :] += part

                @pl.when((k == n_chunks - 1) & (k > 0))
                def _():
                    out_ref[rows, :] += part + b2_ref[0]


@jax.jit
def kernel(x, Wr, br, W1, b1, W2, b2):
    N, D = x.shape
    E, F, _ = W1.shape
    T, K, SB = _T, _K, _SB
    Fc = F // K
    G = N // T + E

    slot2, gate2, te2, nv2 = pl.pallas_call(
        functools.partial(_router_kernel, n_tiles=G, tile=T),
        out_shape=(
            jax.ShapeDtypeStruct((N, 1), jnp.int32),
            jax.ShapeDtypeStruct((N, 1), jnp.float32),
            jax.ShapeDtypeStruct((G, 1), jnp.int32),
            jax.ShapeDtypeStruct((G, 1), jnp.int32),
        ),
    )(x, Wr, br.reshape(1, E))
    slot = slot2[:, 0]
    te = te2[:, 0]
    nv = nv2[:, 0]

    src = jnp.zeros((G * T,), jnp.int32).at[slot].set(
        jnp.arange(N, dtype=jnp.int32), unique_indices=True)
    xp = x[src]

    def _im_x(i, k, te, nv):
        return (jnp.where(nv[i] > 0, i, 0), 0)

    def _im_w1(i, k, te, nv):
        return (te[i], jnp.where(nv[i] > 0, k, K - 1), 0)

    def _im_b1(i, k, te, nv):
        return (te[i], 0, jnp.where(nv[i] > 0, k, K - 1))

    def _im_w2(i, k, te, nv):
        return (te[i], 0, jnp.where(nv[i] > 0, k, K - 1))

    def _im_b2(i, k, te, nv):
        return (te[i], 0, 0)

    grid_spec = pltpu.PrefetchScalarGridSpec(
        num_scalar_prefetch=2,
        grid=(G, K),
        in_specs=[
            pl.BlockSpec((T, D), _im_x),
            pl.BlockSpec((1, Fc, D), _im_w1),
            pl.BlockSpec((1, 1, Fc), _im_b1),
            pl.BlockSpec((1, D, Fc), _im_w2),
            pl.BlockSpec((1, 1, D), _im_b2),
        ],
        out_specs=pl.BlockSpec((T, D), lambda i, k, te, nv: (i, 0)),
    )
    yp = pl.pallas_call(
        functools.partial(_ffn_kernel, n_chunks=K, tile=T, sub=SB),
        grid_spec=grid_spec,
        out_shape=jax.ShapeDtypeStruct((G * T, D), jnp.float32),
        compiler_params=pltpu.CompilerParams(
            vmem_limit_bytes=62 * 1024 * 1024),
    )(te, nv, xp, W1, b1.reshape(E, 1, F), W2, b2.reshape(E, 1, D))

    return yp[slot] * gate2


# speedup vs baseline: 1.1630x; 1.1630x over previous
import functools

import jax
import jax.numpy as jnp
from jax.experimental import pallas as pl
from jax.experimental.pallas import tpu as pltpu

_T = 512
_K = 2
_SB = 256


def _router_kernel(x_ref, wr_ref, br_ref, slot_ref, gate_ref, te_ref,
                   nv_ref, *, n_tiles, tile):
    N = x_ref.shape[0]
    E = wr_ref.shape[0]
    G, T = n_tiles, tile
    x = x_ref[...]
    wr = wr_ref[...]
    logits = jax.lax.dot_general(
        x, wr, (((1,), (1,)), ((), ())), preferred_element_type=jnp.float32)
    logits = logits + br_ref[...]
    m = jnp.max(logits, axis=1, keepdims=True)
    s = jnp.sum(jnp.exp(logits - m), axis=1, keepdims=True)
    gate_ref[...] = 1.0 / s
    idx = jnp.argmax(logits, axis=1).astype(jnp.int32)

    lane_e = jax.lax.broadcasted_iota(jnp.int32, (N, E), 1)
    oh = (lane_e == idx[:, None]).astype(jnp.float32)

    r = jax.lax.broadcasted_iota(jnp.int32, (N, N), 0)
    c = jax.lax.broadcasted_iota(jnp.int32, (N, N), 1)
    tri = (r > c).astype(jnp.float32)
    pos_e = jax.lax.dot_general(
        tri, oh, (((1,), (0,)), ((), ())),
        preferred_element_type=jnp.float32)
    pos = jnp.sum(oh * pos_e, axis=1, keepdims=True)

    counts = jnp.sum(oh, axis=0, keepdims=True)
    tiles_e = jnp.floor((counts + (T - 1)) * (1.0 / T))
    ue = jax.lax.broadcasted_iota(jnp.int32, (E, E), 0)
    uc = jax.lax.broadcasted_iota(jnp.int32, (E, E), 1)
    ut = (ue <= uc).astype(jnp.float32)
    cum_tiles = jax.lax.dot_general(
        tiles_e, ut, (((1,), (0,)), ((), ())),
        preferred_element_type=jnp.float32)
    tile_row_off = (cum_tiles - tiles_e) * T
    slot = jnp.sum(oh * tile_row_off, axis=1, keepdims=True) + pos
    slot_ref[...] = slot.astype(jnp.int32)

    total_tiles = jnp.sum(tiles_e)
    lane_e1 = jax.lax.broadcasted_iota(jnp.int32, (1, E), 1).astype(jnp.float32)
    last_e = jnp.max(lane_e1 * (counts > 0))
    gi = jax.lax.broadcasted_iota(jnp.int32, (G, 1), 0).astype(jnp.float32)
    te_raw = jnp.sum((cum_tiles <= gi).astype(jnp.float32),
                     axis=1, keepdims=True)
    te = jnp.minimum(te_raw, last_e)
    te_ref[...] = te.astype(jnp.int32)

    oh_te = (te == jax.lax.broadcasted_iota(jnp.int32, (G, E), 1)
             .astype(jnp.float32)).astype(jnp.float32)
    cum_te = jnp.sum(oh_te * cum_tiles, axis=1, keepdims=True)
    tiles_te = jnp.sum(oh_te * tiles_e, axis=1, keepdims=True)
    counts_te = jnp.sum(oh_te * counts, axis=1, keepdims=True)
    local_t = gi - (cum_te - tiles_te)
    nv = jnp.clip(counts_te - local_t * T, 0.0, float(T))
    nv = jnp.where(gi < total_tiles, nv, 0.0)
    nv_ref[...] = nv.astype(jnp.int32)


def _ffn_kernel(te_ref, nv_ref, xp_ref, w1_ref, b1_ref, w2_ref, b2_ref,
                out_ref, *, n_chunks, tile, sub):
    del te_ref
    i = pl.program_id(0)
    k = pl.program_id(1)
    nv = nv_ref[i]

    @pl.when(nv > 0)
    def _():
        w1 = w1_ref[0].astype(jnp.bfloat16)
        w2 = w2_ref[0].astype(jnp.bfloat16)
        for sb in range(tile // sub):
            rows = pl.ds(sb * sub, sub)

            @pl.when(sb * sub < nv)
            def _():
                xb = xp_ref[rows, :].astype(jnp.bfloat16)
                h = jax.lax.dot_general(
                    xb, w1, (((1,), (1,)), ((), ())),
                    preferred_element_type=jnp.float32)
                h = jnp.maximum(h + b1_ref[0], 0.0).astype(jnp.bfloat16)
                part = jax.lax.dot_general(
                    h, w2, (((1,), (1,)), ((), ())),
                    preferred_element_type=jnp.float32)

                @pl.when(k == 0)
                def _():
                    out_ref[rows, :] = part

                @pl.when((k > 0) & (k < n_chunks - 1))
                def _():
                    out_ref[rows, :] += part

                @pl.when((k == n_chunks - 1) & (k > 0))
                def _():
                    out_ref[rows, :] += part + b2_ref[0]


@jax.jit
def kernel(x, Wr, br, W1, b1, W2, b2):
    N, D = x.shape
    E, F, _ = W1.shape
    T, K, SB = _T, _K, _SB
    Fc = F // K
    G = N // T + E

    slot2, gate2, te2, nv2 = pl.pallas_call(
        functools.partial(_router_kernel, n_tiles=G, tile=T),
        out_shape=(
            jax.ShapeDtypeStruct((N, 1), jnp.int32),
            jax.ShapeDtypeStruct((N, 1), jnp.float32),
            jax.ShapeDtypeStruct((G, 1), jnp.int32),
            jax.ShapeDtypeStruct((G, 1), jnp.int32),
        ),
    )(x, Wr, br.reshape(1, E))
    slot = slot2[:, 0]
    te = te2[:, 0]
    nv = nv2[:, 0]

    xp = jnp.zeros((G * T, D), x.dtype).at[slot].set(x, unique_indices=True)

    def _im_x(i, k, te, nv):
        return (jnp.where(nv[i] > 0, i, 0), 0)

    def _im_w1(i, k, te, nv):
        return (te[i], jnp.where(nv[i] > 0, k, K - 1), 0)

    def _im_b1(i, k, te, nv):
        return (te[i], 0, jnp.where(nv[i] > 0, k, K - 1))

    def _im_w2(i, k, te, nv):
        return (te[i], 0, jnp.where(nv[i] > 0, k, K - 1))

    def _im_b2(i, k, te, nv):
        return (te[i], 0, 0)

    grid_spec = pltpu.PrefetchScalarGridSpec(
        num_scalar_prefetch=2,
        grid=(G, K),
        in_specs=[
            pl.BlockSpec((T, D), _im_x),
            pl.BlockSpec((1, Fc, D), _im_w1),
            pl.BlockSpec((1, 1, Fc), _im_b1),
            pl.BlockSpec((1, D, Fc), _im_w2),
            pl.BlockSpec((1, 1, D), _im_b2),
        ],
        out_specs=pl.BlockSpec((T, D), lambda i, k, te, nv: (i, 0)),
    )
    yp = pl.pallas_call(
        functools.partial(_ffn_kernel, n_chunks=K, tile=T, sub=SB),
        grid_spec=grid_spec,
        out_shape=jax.ShapeDtypeStruct((G * T, D), jnp.float32),
        compiler_params=pltpu.CompilerParams(
            vmem_limit_bytes=62 * 1024 * 1024),
    )(te, nv, xp, W1, b1.reshape(E, 1, F), W2, b2.reshape(E, 1, D))

    return yp[slot] * gate2
